# Initial kernel scaffold; baseline (speedup 1.0000x reference)
import jax
import jax.numpy as jnp
from jax.experimental import pallas as pl


def kernel(x, edge_index, edge_weight, W1, b1, W2, b2, W3, b3, Wout, bout):
    raise NotImplementedError("write your pallas kernel here")


# speedup vs baseline: 17.4567x; 17.4567x over previous
import functools

import jax
import jax.numpy as jnp
from jax import lax
from jax.experimental import pallas as pl
from jax.experimental.pallas import tpu as pltpu
from jax.experimental.pallas import tpu_sc as plsc

N = 100000
D = 32
DH = 16
NC, NS = 2, 16
E = 1600000
K = 1024
KR = K // 128
EP = 16384 * 98
EROWS = EP // 128
AGG_ROWS_PER_SUB = EROWS // NS
AGG_WINDOWS = AGG_ROWS_PER_SUB // KR
DEG_ROWS_PER_W = EROWS // (NC * NS)
DEG_WINDOWS = DEG_ROWS_PER_W // KR
NPAD = 104448
NROWS_PER_SUB = NPAD // NS
ZCHUNKS = (1024, 1024, 1024, 1024, 1024, 1024, 384)
BT = 1024
F32 = jnp.float32


def _sc_mesh():
  return plsc.VectorSubcoreMesh(
      core_axis_name="c", subcore_axis_name="s",
      num_cores=NC, num_subcores=NS)


@functools.cache
def _deg_kernel():
  @functools.partial(
      pl.kernel,
      mesh=_sc_mesh(),
      out_type=jax.ShapeDtypeStruct((NC, NPAD), F32),
      compiler_params=pltpu.CompilerParams(use_tc_tiling_on_sc=False),
      scratch_types=[
          pltpu.VMEM((KR, 128), jnp.int32),
          pltpu.VMEM((KR, 128), F32),
          pltpu.VMEM((1024,), F32),
          pltpu.VMEM_SHARED((NPAD,), F32),
      ],
  )
  def deg(dst_hbm, ew_hbm, out_hbm, dstb, ewb, zb, acc):
    c = lax.axis_index("c")
    s = lax.axis_index("s")
    w = s * NC + c

    @plsc.parallel_loop(0, 1024 // 16)
    def _zero(i):
      zb[pl.ds(i * 16, 16)] = jnp.zeros((16,), F32)

    off = 0
    for ch in ZCHUNKS:
      pltpu.sync_copy(zb.at[pl.ds(0, ch)],
                      acc.at[pl.ds(s * NROWS_PER_SUB + off, ch)])
      off += ch
    plsc.subcore_barrier()

    def win(wi, _):
      base = w * DEG_ROWS_PER_W + wi * KR
      pltpu.sync_copy(dst_hbm.at[pl.ds(base, KR)], dstb)
      pltpu.sync_copy(ew_hbm.at[pl.ds(base, KR)], ewb)
      for j in range(KR):
        pltpu.sync_copy(ewb.at[j], acc.at[dstb.at[j]], add=True)
      return ()

    lax.fori_loop(0, DEG_WINDOWS, win, ())
    plsc.subcore_barrier()
    pltpu.sync_copy(
        acc.at[pl.ds(s * NROWS_PER_SUB, NROWS_PER_SUB)],
        out_hbm.at[c, pl.ds(s * NROWS_PER_SUB, NROWS_PER_SUB)])

  return deg


@functools.cache
def _agg_kernel():
  @functools.partial(
      pl.kernel,
      mesh=_sc_mesh(),
      out_type=jax.ShapeDtypeStruct((NC, NPAD, DH), F32),
      compiler_params=pltpu.CompilerParams(use_tc_tiling_on_sc=False),
      scratch_types=[
          pltpu.VMEM((KR, 128), jnp.int32),
          pltpu.VMEM((KR, 128), jnp.int32),
          pltpu.VMEM((KR, 128), F32),
          pltpu.VMEM((K, DH), F32),
          pltpu.VMEM_SHARED((NPAD, DH), F32),
          pltpu.SemaphoreType.DMA,
      ],
  )
  def agg(h_hbm, src_hbm, dst_hbm, ew_hbm, out_hbm,
          srcb, dstb, ewb, rowsb, acc, sem):
    c = lax.axis_index("c")
    s = lax.axis_index("s")

    @plsc.parallel_loop(0, K)
    def _zero(i):
      rowsb[i, :] = jnp.zeros((DH,), F32)

    off = 0
    for ch in ZCHUNKS:
      pltpu.sync_copy(
          rowsb.at[pl.ds(0, ch)],
          acc.at[pl.ds(s * NROWS_PER_SUB + off, ch)])
      off += ch
    plsc.subcore_barrier()

    hview = h_hbm.at[c]

    def win(wi, _):
      base = s * AGG_ROWS_PER_SUB + wi * KR
      pltpu.sync_copy(src_hbm.at[pl.ds(base, KR)], srcb)
      pltpu.sync_copy(dst_hbm.at[pl.ds(base, KR)], dstb)
      pltpu.sync_copy(ew_hbm.at[pl.ds(base, KR)], ewb)
      descs = [
          pltpu.async_copy(
              hview.at[srcb.at[j]], rowsb.at[pl.ds(j * 128, 128)], sem)
          for j in range(KR)
      ]
      for d in descs:
        d.wait()

      @plsc.parallel_loop(0, K // 16, unroll=2)
      def _scale(g):
        ev = ewb[lax.shift_right_logical(g, 3),
                 pl.ds(lax.bitwise_and(g, 7) * 16, 16)]
        base = g * 16
        for t in range(16):
          rowsb[base + t, :] = rowsb[base + t, :] * ev[t]

      for j in range(KR):
        pltpu.sync_copy(
            rowsb.at[pl.ds(j * 128, 128)], acc.at[dstb.at[j]], add=True)
      return ()

    lax.fori_loop(0, AGG_WINDOWS, win, ())
    plsc.subcore_barrier()
    pltpu.sync_copy(
        acc.at[pl.ds(s * NROWS_PER_SUB, NROWS_PER_SUB)],
        out_hbm.at[c, pl.ds(s * NROWS_PER_SUB, NROWS_PER_SUB)])

  return agg


def _row_spec(shape):
  nd = len(shape)
  if nd == 2:
    return pl.BlockSpec((BT, shape[1]), lambda i: (i, 0))
  return pl.BlockSpec((shape[0], BT, shape[2]), lambda i: (0, i, 0))


def _full_spec(shape):
  nd = len(shape)
  return pl.BlockSpec(shape, lambda i: (0,) * nd)


def _dense_pre(x_p, d0, d1, w1):
  def body(x_ref, d0_ref, d1_ref, w_ref, h_ref, di_ref):
    deg = d0_ref[...] + d1_ref[...] + 1.0
    di = lax.rsqrt(deg)
    h = jnp.dot(x_ref[...], w_ref[...], preferred_element_type=F32)
    hp = h * di
    h_ref[0] = hp[:, :DH]
    h_ref[1] = hp[:, DH:]
    di_ref[...] = di

  return pl.pallas_call(
      body,
      grid=(NPAD // BT,),
      in_specs=[
          _row_spec((NPAD, D)),
          _row_spec((NPAD, 1)),
          _row_spec((NPAD, 1)),
          _full_spec((D, D)),
      ],
      out_specs=[
          _row_spec((NC, NPAD, DH)),
          _row_spec((NPAD, 1)),
      ],
      out_shape=[
          jax.ShapeDtypeStruct((NC, NPAD, DH), F32),
          jax.ShapeDtypeStruct((NPAD, 1), F32),
      ],
  )(x_p, d0, d1, w1)


def _dense_mid(acc, hp, dinv, b, w_next):
  def body(a_ref, h_ref, di_ref, b_ref, w_ref, o_ref):
    di = di_ref[...]
    a = jnp.concatenate([a_ref[0], a_ref[1]], axis=1)
    h = jnp.concatenate([h_ref[0], h_ref[1]], axis=1)
    o = jnp.maximum(di * (a + h) + b_ref[...], 0.0)
    hn = jnp.dot(o, w_ref[...], preferred_element_type=F32) * di
    o_ref[0] = hn[:, :DH]
    o_ref[1] = hn[:, DH:]

  return pl.pallas_call(
      body,
      grid=(NPAD // BT,),
      in_specs=[
          _row_spec((NC, NPAD, DH)),
          _row_spec((NC, NPAD, DH)),
          _row_spec((NPAD, 1)),
          _full_spec((1, D)),
          _full_spec((D, D)),
      ],
      out_specs=_row_spec((NC, NPAD, DH)),
      out_shape=jax.ShapeDtypeStruct((NC, NPAD, DH), F32),
  )(acc, hp, dinv, b, w_next)


def _dense_fin(acc, hp, dinv, b3, wout, bout):
  def body(a_ref, h_ref, di_ref, b_ref, w_ref, bo_ref, o_ref):
    di = di_ref[...]
    a = jnp.concatenate([a_ref[0], a_ref[1]], axis=1)
    h = jnp.concatenate([h_ref[0], h_ref[1]], axis=1)
    o = jnp.maximum(di * (a + h) + b_ref[...], 0.0)
    z = jnp.dot(o, w_ref[...], preferred_element_type=F32) + bo_ref[...]
    o_ref[...] = jax.nn.sigmoid(z)

  return pl.pallas_call(
      body,
      grid=(NPAD // BT,),
      in_specs=[
          _row_spec((NC, NPAD, DH)),
          _row_spec((NC, NPAD, DH)),
          _row_spec((NPAD, 1)),
          _full_spec((1, D)),
          _full_spec((D, 1)),
          _full_spec((1, 1)),
      ],
      out_specs=_row_spec((NPAD, 1)),
      out_shape=jax.ShapeDtypeStruct((NPAD, 1), F32),
  )(acc, hp, dinv, b3, wout, bout)


def kernel(x, edge_index, edge_weight, W1, b1, W2, b2, W3, b3, Wout, bout):
  pad = EP - E
  padidx = (jnp.arange(pad, dtype=jnp.int32) * 61) % N
  src_p = jnp.concatenate([edge_index[0], padidx]).reshape(EROWS, 128)
  dst_p = jnp.concatenate([edge_index[1], padidx]).reshape(EROWS, 128)
  ew_p = jnp.concatenate(
      [edge_weight, jnp.zeros((pad,), F32)]).reshape(EROWS, 128)
  x_p = jnp.pad(x, ((0, NPAD - N), (0, 0)))

  degp = _deg_kernel()(dst_p, ew_p)
  d0 = degp[0].reshape(NPAD, 1)
  d1 = degp[1].reshape(NPAD, 1)

  h1, dinv = _dense_pre(x_p, d0, d1, W1)
  a1 = _agg_kernel()(h1, src_p, dst_p, ew_p)
  h2 = _dense_mid(a1, h1, dinv, b1.reshape(1, D), W2)
  a2 = _agg_kernel()(h2, src_p, dst_p, ew_p)
  h3 = _dense_mid(a2, h2, dinv, b2.reshape(1, D), W3)
  a3 = _agg_kernel()(h3, src_p, dst_p, ew_p)
  z = _dense_fin(a3, h3, dinv, b3.reshape(1, D), Wout, bout.reshape(1, 1))
  return z[:N]


# speedup vs baseline: 22.6788x; 1.2991x over previous
import functools

import jax
import jax.numpy as jnp
from jax import lax
from jax.experimental import pallas as pl
from jax.experimental.pallas import tpu as pltpu
from jax.experimental.pallas import tpu_sc as plsc

N = 100000
D = 32
DH = 16
NC, NS = 2, 16
E = 1600000
K = 512
KR = K // 128
KD = 1024
KDR = KD // 128
EP = 16384 * 98
EROWS = EP // 128
AGG_ROWS_PER_SUB = EROWS // NS
AGG_WINDOWS = AGG_ROWS_PER_SUB // KR
DEG_ROWS_PER_W = EROWS // (NC * NS)
DEG_WINDOWS = DEG_ROWS_PER_W // KDR
NPAD = 104448
NROWS_PER_SUB = NPAD // NS
ZCHUNKS = (1024, 1024, 1024, 1024, 1024, 1024, 384)
AZCHUNKS = tuple([K] * (NROWS_PER_SUB // K)
                 + ([NROWS_PER_SUB % K] if NROWS_PER_SUB % K else []))
BT = 1024
F32 = jnp.float32


def _sc_mesh():
  return plsc.VectorSubcoreMesh(
      core_axis_name="c", subcore_axis_name="s",
      num_cores=NC, num_subcores=NS)


@functools.cache
def _deg_kernel():
  @functools.partial(
      pl.kernel,
      mesh=_sc_mesh(),
      out_type=jax.ShapeDtypeStruct((NC, NPAD), F32),
      compiler_params=pltpu.CompilerParams(use_tc_tiling_on_sc=False),
      scratch_types=[
          pltpu.VMEM((KDR, 128), jnp.int32),
          pltpu.VMEM((KDR, 128), F32),
          pltpu.VMEM((1024,), F32),
          pltpu.VMEM_SHARED((NPAD,), F32),
      ],
  )
  def deg(dst_hbm, ew_hbm, out_hbm, dstb, ewb, zb, acc):
    c = lax.axis_index("c")
    s = lax.axis_index("s")
    w = s * NC + c

    @plsc.parallel_loop(0, 1024 // 16)
    def _zero(i):
      zb[pl.ds(i * 16, 16)] = jnp.zeros((16,), F32)

    off = 0
    for ch in ZCHUNKS:
      pltpu.sync_copy(zb.at[pl.ds(0, ch)],
                      acc.at[pl.ds(s * NROWS_PER_SUB + off, ch)])
      off += ch
    plsc.subcore_barrier()

    def win(wi, _):
      base = w * DEG_ROWS_PER_W + wi * KDR
      pltpu.sync_copy(dst_hbm.at[pl.ds(base, KDR)], dstb)
      pltpu.sync_copy(ew_hbm.at[pl.ds(base, KDR)], ewb)
      for j in range(KDR):
        pltpu.sync_copy(ewb.at[j], acc.at[dstb.at[j]], add=True)
      return ()

    lax.fori_loop(0, DEG_WINDOWS, win, ())
    plsc.subcore_barrier()
    pltpu.sync_copy(
        acc.at[pl.ds(s * NROWS_PER_SUB, NROWS_PER_SUB)],
        out_hbm.at[c, pl.ds(s * NROWS_PER_SUB, NROWS_PER_SUB)])

  return deg


@functools.cache
def _agg_kernel():
  @functools.partial(
      pl.kernel,
      mesh=_sc_mesh(),
      out_type=jax.ShapeDtypeStruct((NC, NPAD, DH), F32),
      compiler_params=pltpu.CompilerParams(use_tc_tiling_on_sc=False),
      scratch_types=[
          pltpu.VMEM((KR, 128), jnp.int32),
          pltpu.VMEM((KR, 128), jnp.int32),
          pltpu.VMEM((KR, 128), jnp.int32),
          pltpu.VMEM((KR, 128), jnp.int32),
          pltpu.VMEM((KR, 128), F32),
          pltpu.VMEM((KR, 128), F32),
          pltpu.VMEM((K, DH), F32),
          pltpu.VMEM((K, DH), F32),
          pltpu.VMEM_SHARED((NPAD, DH), F32),
          pltpu.SemaphoreType.DMA,
          pltpu.SemaphoreType.DMA,
          pltpu.SemaphoreType.DMA,
          pltpu.SemaphoreType.DMA,
      ],
  )
  def agg(h_hbm, src_hbm, dst_hbm, ew_hbm, out_hbm,
          srcb0, srcb1, dstb0, dstb1, ewb0, ewb1, rows0, rows1, acc,
          gsem0, gsem1, isem0, isem1):
    c = lax.axis_index("c")
    s = lax.axis_index("s")
    srcb = (srcb0, srcb1)
    dstb = (dstb0, dstb1)
    ewb = (ewb0, ewb1)
    rows = (rows0, rows1)
    gsem = (gsem0, gsem1)
    isem = (isem0, isem1)

    @plsc.parallel_loop(0, K)
    def _zero(i):
      rows0[i, :] = jnp.zeros((DH,), F32)

    off = 0
    for ch in AZCHUNKS:
      pltpu.sync_copy(
          rows0.at[pl.ds(0, ch)],
          acc.at[pl.ds(s * NROWS_PER_SUB + off, ch)])
      off += ch
    plsc.subcore_barrier()

    hview = h_hbm.at[c]
    sub_base = s * AGG_ROWS_PER_SUB

    def load_idx(w, p):
      base = sub_base + w * KR
      pltpu.async_copy(src_hbm.at[pl.ds(base, KR)], srcb[p], isem[p])
      pltpu.async_copy(dst_hbm.at[pl.ds(base, KR)], dstb[p], isem[p])
      pltpu.async_copy(ew_hbm.at[pl.ds(base, KR)], ewb[p], isem[p])

    def drain_idx(p):
      base = sub_base
      pltpu.make_async_copy(src_hbm.at[pl.ds(base, KR)], srcb[p],
                            isem[p]).wait()
      pltpu.make_async_copy(dst_hbm.at[pl.ds(base, KR)], dstb[p],
                            isem[p]).wait()
      pltpu.make_async_copy(ew_hbm.at[pl.ds(base, KR)], ewb[p],
                            isem[p]).wait()

    def fire_gather(p):
      for j in range(KR):
        pltpu.async_copy(
            hview.at[srcb[p].at[j]], rows[p].at[pl.ds(j * 128, 128)],
            gsem[p])

    def drain_gather(p):
      pltpu.make_async_copy(
          hview.at[pl.ds(0, K)], rows[p], gsem[p]).wait()

    def scale_scatter(p):
      ewp, rp, dp = ewb[p], rows[p], dstb[p]

      @plsc.parallel_loop(0, K // 16, unroll=2)
      def _scale(g):
        ev = ewp[lax.shift_right_logical(g, 3),
                 pl.ds(lax.bitwise_and(g, 7) * 16, 16)]
        base = g * 16
        for t in range(16):
          rp[base + t, :] = rp[base + t, :] * ev[t]

      for j in range(KR):
        pltpu.sync_copy(
            rp.at[pl.ds(j * 128, 128)], acc.at[dp.at[j]], add=True)

    load_idx(0, 0)
    drain_idx(0)
    fire_gather(0)
    load_idx(1, 1)

    def pair(g, _):
      drain_idx(1)
      fire_gather(1)
      drain_gather(0)
      scale_scatter(0)

      @pl.when(g < AGG_WINDOWS // 2 - 1)
      def _prefetch_even():
        load_idx(2 * g + 2, 0)

      drain_gather(1)
      scale_scatter(1)

      @pl.when(g < AGG_WINDOWS // 2 - 1)
      def _next_gather():
        drain_idx(0)
        fire_gather(0)
        load_idx(2 * g + 3, 1)

      return ()

    lax.fori_loop(0, AGG_WINDOWS // 2, pair, ())
    plsc.subcore_barrier()
    pltpu.sync_copy(
        acc.at[pl.ds(s * NROWS_PER_SUB, NROWS_PER_SUB)],
        out_hbm.at[c, pl.ds(s * NROWS_PER_SUB, NROWS_PER_SUB)])

  return agg


def _row_spec(shape):
  nd = len(shape)
  if nd == 2:
    return pl.BlockSpec((BT, shape[1]), lambda i: (i, 0))
  return pl.BlockSpec((shape[0], BT, shape[2]), lambda i: (0, i, 0))


def _full_spec(shape):
  nd = len(shape)
  return pl.BlockSpec(shape, lambda i: (0,) * nd)


def _dense_pre(x_p, d0, d1, w1):
  def body(x_ref, d0_ref, d1_ref, w_ref, h_ref, di_ref):
    deg = d0_ref[...] + d1_ref[...] + 1.0
    di = lax.rsqrt(deg)
    h = jnp.dot(x_ref[...], w_ref[...], preferred_element_type=F32)
    hp = h * di
    h_ref[0] = hp[:, :DH]
    h_ref[1] = hp[:, DH:]
    di_ref[...] = di

  return pl.pallas_call(
      body,
      grid=(NPAD // BT,),
      in_specs=[
          _row_spec((NPAD, D)),
          _row_spec((NPAD, 1)),
          _row_spec((NPAD, 1)),
          _full_spec((D, D)),
      ],
      out_specs=[
          _row_spec((NC, NPAD, DH)),
          _row_spec((NPAD, 1)),
      ],
      out_shape=[
          jax.ShapeDtypeStruct((NC, NPAD, DH), F32),
          jax.ShapeDtypeStruct((NPAD, 1), F32),
      ],
  )(x_p, d0, d1, w1)


def _dense_mid(acc, hp, dinv, b, w_next):
  def body(a_ref, h_ref, di_ref, b_ref, w_ref, o_ref):
    di = di_ref[...]
    a = jnp.concatenate([a_ref[0], a_ref[1]], axis=1)
    h = jnp.concatenate([h_ref[0], h_ref[1]], axis=1)
    o = jnp.maximum(di * (a + h) + b_ref[...], 0.0)
    hn = jnp.dot(o, w_ref[...], preferred_element_type=F32) * di
    o_ref[0] = hn[:, :DH]
    o_ref[1] = hn[:, DH:]

  return pl.pallas_call(
      body,
      grid=(NPAD // BT,),
      in_specs=[
          _row_spec((NC, NPAD, DH)),
          _row_spec((NC, NPAD, DH)),
          _row_spec((NPAD, 1)),
          _full_spec((1, D)),
          _full_spec((D, D)),
      ],
      out_specs=_row_spec((NC, NPAD, DH)),
      out_shape=jax.ShapeDtypeStruct((NC, NPAD, DH), F32),
  )(acc, hp, dinv, b, w_next)


def _dense_fin(acc, hp, dinv, b3, wout, bout):
  def body(a_ref, h_ref, di_ref, b_ref, w_ref, bo_ref, o_ref):
    di = di_ref[...]
    a = jnp.concatenate([a_ref[0], a_ref[1]], axis=1)
    h = jnp.concatenate([h_ref[0], h_ref[1]], axis=1)
    o = jnp.maximum(di * (a + h) + b_ref[...], 0.0)
    z = jnp.dot(o, w_ref[...], preferred_element_type=F32) + bo_ref[...]
    o_ref[...] = jax.nn.sigmoid(z)

  return pl.pallas_call(
      body,
      grid=(NPAD // BT,),
      in_specs=[
          _row_spec((NC, NPAD, DH)),
          _row_spec((NC, NPAD, DH)),
          _row_spec((NPAD, 1)),
          _full_spec((1, D)),
          _full_spec((D, 1)),
          _full_spec((1, 1)),
      ],
      out_specs=_row_spec((NPAD, 1)),
      out_shape=jax.ShapeDtypeStruct((NPAD, 1), F32),
  )(acc, hp, dinv, b3, wout, bout)


def kernel(x, edge_index, edge_weight, W1, b1, W2, b2, W3, b3, Wout, bout):
  pad = EP - E
  padidx = (jnp.arange(pad, dtype=jnp.int32) * 61) % N
  src_p = jnp.concatenate([edge_index[0], padidx]).reshape(EROWS, 128)
  dst_p = jnp.concatenate([edge_index[1], padidx]).reshape(EROWS, 128)
  ew_p = jnp.concatenate(
      [edge_weight, jnp.zeros((pad,), F32)]).reshape(EROWS, 128)
  x_p = jnp.pad(x, ((0, NPAD - N), (0, 0)))

  degp = _deg_kernel()(dst_p, ew_p)
  d0 = degp[0].reshape(NPAD, 1)
  d1 = degp[1].reshape(NPAD, 1)

  h1, dinv = _dense_pre(x_p, d0, d1, W1)
  a1 = _agg_kernel()(h1, src_p, dst_p, ew_p)
  h2 = _dense_mid(a1, h1, dinv, b1.reshape(1, D), W2)
  a2 = _agg_kernel()(h2, src_p, dst_p, ew_p)
  h3 = _dense_mid(a2, h2, dinv, b2.reshape(1, D), W3)
  a3 = _agg_kernel()(h3, src_p, dst_p, ew_p)
  z = _dense_fin(a3, h3, dinv, b3.reshape(1, D), Wout, bout.reshape(1, 1))
  return z[:N]
